# Initial kernel scaffold; baseline (speedup 1.0000x reference)
import jax
import jax.numpy as jnp
from jax import lax
from jax.experimental import pallas as pl
from jax.experimental.pallas import tpu as pltpu
from jax.experimental.pallas import tpu_sc as plsc

NC, NS = 2, 16
NW = NC * NS
B = 4096 * 200
D = 32
GROUP = 128
GPW = B // (NW * GROUP)
K = 8
T = GPW // K


def _emb_body(idx_hbm, w_hbm, out_hbm, idx_v, rows_v, sem):
    wid = lax.axis_index("s") * NC + lax.axis_index("c")
    g0 = wid * GPW
    pltpu.sync_copy(idx_hbm.at[pl.ds(g0, GPW)], idx_v)

    @pl.loop(0, T)
    def _step(t):
        descs = []
        for b in range(K):
            d = pltpu.async_copy(
                w_hbm.at[idx_v.at[t * K + b]],
                rows_v.at[pl.ds(b * GROUP, GROUP)],
                sem,
            )
            descs.append(d)
        for d in descs:
            d.wait()
        pltpu.sync_copy(
            rows_v, out_hbm.at[pl.ds((g0 + t * K) * GROUP, K * GROUP)]
        )


def kernel(x, W):
    idx = x.reshape(B // GROUP, GROUP).astype(jnp.int32)
    out = pl.kernel(
        _emb_body,
        out_type=jax.ShapeDtypeStruct((B, D), jnp.float32),
        mesh=plsc.VectorSubcoreMesh(core_axis_name="c", subcore_axis_name="s"),
        scratch_types=[
            pltpu.VMEM((GPW, GROUP), jnp.int32),
            pltpu.VMEM((K * GROUP, D), jnp.float32),
            pltpu.SemaphoreType.DMA,
        ],
    )(idx, W)
    return out.reshape(x.shape[0], x.shape[1], D)


# speedup vs baseline: 1.4765x; 1.4765x over previous
import jax
import jax.numpy as jnp
from jax import lax
from jax.experimental import pallas as pl
from jax.experimental.pallas import tpu as pltpu
from jax.experimental.pallas import tpu_sc as plsc

NC, NS = 2, 16
NW = NC * NS
B = 4096 * 200
D = 32
GROUP = 128
GPW = B // (NW * GROUP)
K = 8
T = GPW // K


def _emb_body(idx_hbm, w_hbm, out_hbm, idx_v, rows_v, sem):
    wid = lax.axis_index("s") * NC + lax.axis_index("c")
    g0 = wid * GPW
    pltpu.sync_copy(idx_hbm.at[pl.ds(g0, GPW)], idx_v)

    @pl.loop(0, T)
    def _step(t):
        descs = []
        for b in range(K):
            d = pltpu.async_copy(
                w_hbm.at[idx_v.at[t * K + b]],
                rows_v.at[pl.ds(b * GROUP, GROUP)],
                sem,
            )
            descs.append(d)
        for d in descs:
            d.wait()
        pltpu.sync_copy(
            rows_v, out_hbm.at[pl.ds((g0 + t * K) * GROUP, K * GROUP)]
        )


def kernel(x, W):
    idx = x.reshape(B // GROUP, GROUP).astype(jnp.int32)
    out = pl.kernel(
        _emb_body,
        out_type=jax.ShapeDtypeStruct((B, D), jnp.float32),
        mesh=plsc.VectorSubcoreMesh(core_axis_name="c", subcore_axis_name="s"),
        compiler_params=pltpu.CompilerParams(use_tc_tiling_on_sc=False),
        scratch_types=[
            pltpu.VMEM((GPW, GROUP), jnp.int32),
            pltpu.VMEM((K * GROUP, D), jnp.float32),
            pltpu.SemaphoreType.DMA,
        ],
    )(idx, W)
    return out.reshape(x.shape[0], x.shape[1], D)


# speedup vs baseline: 1.5023x; 1.0175x over previous
import jax
import jax.numpy as jnp
from jax import lax
from jax.experimental import pallas as pl
from jax.experimental.pallas import tpu as pltpu
from jax.experimental.pallas import tpu_sc as plsc

NC, NS = 2, 16
NW = NC * NS
B = 4096 * 200
D = 32
GROUP = 128
GPW = B // (NW * GROUP)
K = 5
CH = K * GROUP
NBUF = 4
T = GPW // K


def _emb_body(idx_hbm, w_hbm, out_hbm, idx_v, rows_v, gsem, *osems):
    wid = lax.axis_index("s") * NC + lax.axis_index("c")
    g0 = wid * GPW
    pltpu.sync_copy(idx_hbm.at[pl.ds(g0, GPW)], idx_v)

    def fire_g(t, b):
        for j in range(K):
            pltpu.async_copy(
                w_hbm.at[idx_v.at[t * K + j]],
                rows_v.at[b].at[pl.ds(j * GROUP, GROUP)],
                gsem,
            )

    def drain_g(b):
        pltpu.make_async_copy(
            w_hbm.at[pl.ds(0, CH)], rows_v.at[b], gsem
        ).wait()

    def fire_out(t, b):
        pltpu.async_copy(
            rows_v.at[b], out_hbm.at[pl.ds((g0 + t * K) * GROUP, CH)], osems[b]
        )

    def drain_out(b):
        pltpu.make_async_copy(
            rows_v.at[b], out_hbm.at[pl.ds(0, CH)], osems[b]
        ).wait()

    for b in range(NBUF):
        fire_g(b, b)

    @pl.loop(0, T - NBUF, step=NBUF)
    def _steady(tt):
        for b in range(NBUF):
            t = tt + b
            drain_g(b)
            fire_out(t, b)
            drain_out(b)
            fire_g(t + NBUF, b)

    for b in range(NBUF):
        drain_g(b)
        fire_out(T - NBUF + b, b)
    for b in range(NBUF):
        drain_out(b)


def kernel(x, W):
    idx = x.reshape(B // GROUP, GROUP).astype(jnp.int32)
    out = pl.kernel(
        _emb_body,
        out_type=jax.ShapeDtypeStruct((B, D), jnp.float32),
        mesh=plsc.VectorSubcoreMesh(core_axis_name="c", subcore_axis_name="s"),
        compiler_params=pltpu.CompilerParams(use_tc_tiling_on_sc=False),
        scratch_types=[
            pltpu.VMEM((GPW, GROUP), jnp.int32),
            pltpu.VMEM((NBUF, CH, D), jnp.float32),
            pltpu.SemaphoreType.DMA,
        ]
        + [pltpu.SemaphoreType.DMA] * NBUF,
    )(idx, W)
    return out.reshape(x.shape[0], x.shape[1], D)
